# baseline (speedup 1.0000x reference)
import functools

import jax
import jax.numpy as jnp
from jax.experimental import pallas as pl
from jax.experimental.pallas import tpu as pltpu


def _slab_start(i, gamma, *, S, SP, max_len):
    p0 = (i * S).astype(jnp.float32)
    rel0 = jnp.clip(p0 / gamma, 0.0, float(max_len - 1))
    a0 = jnp.floor(rel0).astype(jnp.int32)
    a0 = (a0 // 8) * 8
    a0 = jnp.clip(a0, 0, max_len - SP)
    return pl.multiple_of(a0, 8)


def _body(vel_ref, x_ref, pe_hbm, out_ref, rows_ref, sems, *, S, SP, max_len):
    i = pl.program_id(0)
    n = pl.num_programs(0)
    v = jnp.clip(vel_ref[0], 0.0, 0.999)
    gamma = 1.0 / jnp.sqrt(1.0 - v ** 2)
    start = functools.partial(_slab_start, gamma=gamma, S=S, SP=SP,
                              max_len=max_len)

    @pl.when(i == 0)
    def _prologue():
        pltpu.make_async_copy(pe_hbm.at[pl.ds(start(i), SP)],
                              rows_ref.at[0], sems.at[0]).start()

    @pl.when(i + 1 < n)
    def _prefetch():
        pltpu.make_async_copy(pe_hbm.at[pl.ds(start(i + 1), SP)],
                              rows_ref.at[(i + 1) % 2],
                              sems.at[(i + 1) % 2]).start()

    a0 = start(i)
    t = (i * S).astype(jnp.float32) + jax.lax.broadcasted_iota(
        jnp.int32, (S, 1), 0).astype(jnp.float32)
    rel = jnp.clip(t / gamma, 0.0, float(max_len - 1))
    loc = rel - a0.astype(jnp.float32)
    cols = jax.lax.broadcasted_iota(jnp.int32, (S, SP), 1).astype(jnp.float32)
    w = jnp.maximum(1.0 - jnp.abs(loc - cols), 0.0)

    pltpu.make_async_copy(pe_hbm.at[pl.ds(a0, SP)],
                          rows_ref.at[i % 2], sems.at[i % 2]).wait()
    pe = jax.lax.dot_general(
        w, rows_ref[i % 2], (((1,), (0,)), ((), ())),
        preferred_element_type=jnp.float32)
    out_ref[...] = x_ref[...] + pe[None, :, :]


def kernel(x, velocity, pe_base):
    batch, seq_len, hidden = x.shape
    max_len = pe_base.shape[0]
    S = 128
    SP = S + 8
    body = functools.partial(_body, S=S, SP=SP, max_len=max_len)
    return pl.pallas_call(
        body,
        grid=(seq_len // S,),
        in_specs=[
            pl.BlockSpec(memory_space=pltpu.SMEM),
            pl.BlockSpec((batch, S, hidden), lambda i: (0, i, 0)),
            pl.BlockSpec(memory_space=pltpu.MemorySpace.HBM),
        ],
        out_specs=pl.BlockSpec((batch, S, hidden), lambda i: (0, i, 0)),
        out_shape=jax.ShapeDtypeStruct((batch, seq_len, hidden), x.dtype),
        scratch_shapes=[
            pltpu.VMEM((2, SP, hidden), jnp.float32),
            pltpu.SemaphoreType.DMA((2,)),
        ],
    )(velocity, x, pe_base)


# speedup vs baseline: 1.0785x; 1.0785x over previous
import functools

import jax
import jax.numpy as jnp
from jax.experimental import pallas as pl
from jax.experimental.pallas import tpu as pltpu


def _slab_start(i, gamma, *, S, SP, max_len):
    p0 = (i * S).astype(jnp.float32)
    rel0 = jnp.clip(p0 / gamma, 0.0, float(max_len - 1))
    a0 = jnp.floor(rel0).astype(jnp.int32)
    a0 = (a0 // 8) * 8
    a0 = jnp.clip(a0, 0, max_len - SP)
    return pl.multiple_of(a0, 8)


def _body(vel_ref, x_ref, pe_hbm, out_ref, rows_ref, sems, *, S, SP, max_len):
    i = pl.program_id(0)
    n = pl.num_programs(0)
    v = jnp.clip(vel_ref[0], 0.0, 0.999)
    gamma = 1.0 / jnp.sqrt(1.0 - v ** 2)
    start = functools.partial(_slab_start, gamma=gamma, S=S, SP=SP,
                              max_len=max_len)

    @pl.when(i == 0)
    def _prologue():
        pltpu.make_async_copy(pe_hbm.at[pl.ds(start(i), SP)],
                              rows_ref.at[0], sems.at[0]).start()

    @pl.when(i + 1 < n)
    def _prefetch():
        pltpu.make_async_copy(pe_hbm.at[pl.ds(start(i + 1), SP)],
                              rows_ref.at[(i + 1) % 2],
                              sems.at[(i + 1) % 2]).start()

    a0 = start(i)
    t = (i * S).astype(jnp.float32) + jax.lax.broadcasted_iota(
        jnp.int32, (S, 1), 0).astype(jnp.float32)
    rel = jnp.clip(t / gamma, 0.0, float(max_len - 1))
    loc = rel - a0.astype(jnp.float32)
    cols = jax.lax.broadcasted_iota(jnp.int32, (S, SP), 1).astype(jnp.float32)
    w = jnp.maximum(1.0 - jnp.abs(loc - cols), 0.0)

    pltpu.make_async_copy(pe_hbm.at[pl.ds(a0, SP)],
                          rows_ref.at[i % 2], sems.at[i % 2]).wait()
    pe = jax.lax.dot_general(
        w, rows_ref[i % 2], (((1,), (0,)), ((), ())),
        preferred_element_type=jnp.float32)
    out_ref[...] = x_ref[...] + pe[None, :, :]


def kernel(x, velocity, pe_base):
    batch, seq_len, hidden = x.shape
    max_len = pe_base.shape[0]
    S = 256
    SP = S + 8
    body = functools.partial(_body, S=S, SP=SP, max_len=max_len)
    return pl.pallas_call(
        body,
        grid=(seq_len // S,),
        in_specs=[
            pl.BlockSpec(memory_space=pltpu.SMEM),
            pl.BlockSpec((batch, S, hidden), lambda i: (0, i, 0)),
            pl.BlockSpec(memory_space=pltpu.MemorySpace.HBM),
        ],
        out_specs=pl.BlockSpec((batch, S, hidden), lambda i: (0, i, 0)),
        out_shape=jax.ShapeDtypeStruct((batch, seq_len, hidden), x.dtype),
        scratch_shapes=[
            pltpu.VMEM((2, SP, hidden), jnp.float32),
            pltpu.SemaphoreType.DMA((2,)),
        ],
    )(velocity, x, pe_base)
